# Initial kernel scaffold; baseline (speedup 1.0000x reference)
import jax
import jax.numpy as jnp
from jax.experimental import pallas as pl


def kernel(x, index, weights, gate_w, gate_b, msg_w, msg_b, pow_param):
    raise NotImplementedError("write your pallas kernel here")


# speedup vs baseline: 3.9307x; 3.9307x over previous
import functools

import jax
import jax.numpy as jnp
from jax import lax
from jax.experimental import pallas as pl
from jax.experimental.pallas import tpu as pltpu
from jax.experimental.pallas import tpu_sc as plsc

_N = 320000
_D = 128
_S = 10000
_NW = 32
_SPW = 320
_CH = 1280
_XCH = 256
_TB1 = 3200
_TB4 = 2048
_NEG = -3.0e38


def _prologue_body(x_ref, w_ref, gw_ref, gb_ref, p_ref, gl_ref, wp_ref):
    xv = x_ref[...]
    gw = gw_ref[...]
    gl_ref[...] = jnp.sum(xv * gw, axis=1, keepdims=True) + gb_ref[0]
    wp_ref[...] = jnp.power(w_ref[...], p_ref[0])


def _prologue(x, weights, gate_w_row, gate_b, pow_param):
    grid = (_N // _TB1,)
    return pl.pallas_call(
        _prologue_body,
        grid=grid,
        in_specs=[
            pl.BlockSpec((_TB1, _D), lambda i: (i, 0)),
            pl.BlockSpec((_TB1, 1), lambda i: (i, 0)),
            pl.BlockSpec((1, _D), lambda i: (0, 0)),
            pl.BlockSpec(memory_space=pltpu.SMEM),
            pl.BlockSpec(memory_space=pltpu.SMEM),
        ],
        out_specs=[
            pl.BlockSpec((_TB1, 1), lambda i: (i, 0)),
            pl.BlockSpec((_TB1, 1), lambda i: (i, 0)),
        ],
        out_shape=[
            jax.ShapeDtypeStruct((_N, 1), jnp.float32),
            jax.ShapeDtypeStruct((_N, 1), jnp.float32),
        ],
    )(x, weights, gate_w_row, gate_b, pow_param)


_LANE = 16


def _shift_up(v, k, fill):
    lanes = jax.lax.iota(jnp.int32, _LANE)
    src = jnp.maximum(lanes - k, 0)
    got = jnp.take_along_axis(v, src, axis=0)
    return jnp.where(lanes >= k, got, fill)


def _run_ends(li):
    lanes = jax.lax.iota(jnp.int32, _LANE)
    src = jnp.minimum(lanes + 1, _LANE - 1)
    nxt = jnp.take_along_axis(li, src, axis=0)
    return (nxt != li) | (lanes == _LANE - 1)


def _seg_scan_max(val, li):
    x = val
    for k in (1, 2, 4, 8):
        xs = _shift_up(x, k, _NEG)
        ls = _shift_up(li, k, -1)
        x = jnp.where(ls == li, jnp.maximum(x, xs), x)
    return x


def _seg_scan_sum(val, li):
    x = val
    for k in (1, 2, 4, 8):
        xs = _shift_up(x, k, 0.0)
        ls = _shift_up(li, k, -1)
        x = jnp.where(ls == li, x + xs, x)
    return x


def _sc_body(gl_hbm, wp_hbm, idx_hbm, x_hbm, rs_hbm,
             pooled_hbm, gsn_hbm,
             rsv, idxb, glb, wpb, livb, valb, mbuf, zbuf, gsnb, acc, xb):
    w = lax.axis_index("s") * 2 + lax.axis_index("c")
    s0 = w * _SPW

    pltpu.sync_copy(rs_hbm, rsv)
    wv16 = jnp.full((_LANE,), w, jnp.int32)
    r0 = plsc.load_gather(rsv, [wv16])[0]
    r1 = plsc.load_gather(rsv, [wv16 + 1])[0]
    lo = r0 // _CH
    hi = jnp.maximum((r1 + _CH - 1) // _CH, lo)

    zero16 = jnp.zeros((_LANE,), jnp.float32)
    neg16 = jnp.full((_LANE,), _NEG, jnp.float32)

    def init_small(g, _):
        mbuf[pl.ds(g * 16, 16)] = neg16
        zbuf[pl.ds(g * 16, 16)] = zero16
        gsnb[pl.ds(g * 16, 16)] = zero16
        return 0
    lax.fori_loop(0, _SPW // 16, init_small, 0)

    def init_acc(g, _):
        acc[pl.ds(g * 16, 16)] = zero16
        return 0
    lax.fori_loop(0, _SPW * _D // 16, init_acc, 0)

    def chunk_max(k, _):
        base = k * _CH
        pltpu.sync_copy(gl_hbm.at[pl.ds(base, _CH)], glb)
        pltpu.sync_copy(idx_hbm.at[pl.ds(base, _CH)], idxb)

        def grp(g, _):
            iv = idxb[pl.ds(g * 16, 16)]
            gv = glb[pl.ds(g * 16, 16)]
            own = (iv >= s0) & (iv < s0 + _SPW)
            li = jnp.clip(iv - s0, 0, _SPW - 1)
            val = jnp.where(own, gv, _NEG)
            smax = _seg_scan_max(val, li)
            ends = _run_ends(li)
            cur = plsc.load_gather(mbuf, [li])
            plsc.store_scatter(mbuf, [li], jnp.maximum(cur, smax), mask=ends)
            return 0
        lax.fori_loop(0, _CH // 16, grp, 0)
        return 0
    lax.fori_loop(lo, hi, chunk_max, 0)

    def chunk_acc(k, _):
        base = k * _CH
        pltpu.sync_copy(gl_hbm.at[pl.ds(base, _CH)], glb)
        pltpu.sync_copy(idx_hbm.at[pl.ds(base, _CH)], idxb)
        pltpu.sync_copy(wp_hbm.at[pl.ds(base, _CH)], wpb)

        def grp(g, _):
            iv = idxb[pl.ds(g * 16, 16)]
            gv = glb[pl.ds(g * 16, 16)]
            wvv = wpb[pl.ds(g * 16, 16)]
            own = (iv >= s0) & (iv < s0 + _SPW)
            li = jnp.clip(iv - s0, 0, _SPW - 1)
            mg = plsc.load_gather(mbuf, [li])
            e = jnp.where(own, wvv * jnp.exp(gv - mg), 0.0)
            livb[pl.ds(g * 16, 16)] = li
            valb[pl.ds(g * 16, 16)] = e
            ssum = _seg_scan_sum(e, li)
            ends = _run_ends(li)
            cur = plsc.load_gather(zbuf, [li])
            plsc.store_scatter(zbuf, [li], cur + ssum, mask=ends)
            return 0
        lax.fori_loop(0, _CH // 16, grp, 0)

        def sub(sc, _):
            pltpu.sync_copy(x_hbm.at[pl.ds(base + sc * _XCH, _XCH), :], xb)

            def grpa(g, _):
                li16 = livb[pl.ds(sc * _XCH + g * 16, 16)]
                e16 = valb[pl.ds(sc * _XCH + g * 16, 16)]
                for lane in range(_LANE):
                    li = li16[lane]
                    e = e16[lane]
                    off = li * _D
                    row = g * 16 + lane
                    for j in range(_D // 16):
                        xv = xb[row, pl.ds(j * 16, 16)]
                        plsc.addupdate(acc.at[pl.ds(off + j * 16, 16)], xv * e)
                return 0
            lax.fori_loop(0, _XCH // 16, grpa, 0)
            return 0
        lax.fori_loop(0, _CH // _XCH, sub, 0)
        return 0
    lax.fori_loop(lo, hi, chunk_acc, 0)

    def fin(g, _):
        zv = zbuf[pl.ds(g * 16, 16)]
        inv = 1.0 / (zv + 1e-10)
        gsnb[pl.ds(g * 16, 16)] = zv * inv
        for lane in range(_LANE):
            ivl = inv[lane]
            off = (g * 16 + lane) * _D
            for j in range(_D // 16):
                sl = pl.ds(off + j * 16, 16)
                acc[sl] = acc[sl] * ivl
        return 0
    lax.fori_loop(0, _SPW // 16, fin, 0)

    pltpu.sync_copy(acc, pooled_hbm.at[pl.ds(w * _SPW * _D, _SPW * _D)])
    pltpu.sync_copy(gsnb, gsn_hbm.at[pl.ds(w * _SPW, _SPW)])


def _sc_call(gl, wp, idx, x, rs):
    mesh = plsc.VectorSubcoreMesh(core_axis_name="c", subcore_axis_name="s")
    f = pl.kernel(
        _sc_body,
        compiler_params=pltpu.CompilerParams(needs_layout_passes=False),
        out_type=[
            jax.ShapeDtypeStruct((_NW * _SPW * _D,), jnp.float32),
            jax.ShapeDtypeStruct((_NW * _SPW,), jnp.float32),
        ],
        mesh=mesh,
        scratch_types=[
            pltpu.VMEM((48,), jnp.int32),
            pltpu.VMEM((_CH,), jnp.int32),
            pltpu.VMEM((_CH,), jnp.float32),
            pltpu.VMEM((_CH,), jnp.float32),
            pltpu.VMEM((_CH,), jnp.int32),
            pltpu.VMEM((_CH,), jnp.float32),
            pltpu.VMEM((_SPW,), jnp.float32),
            pltpu.VMEM((_SPW,), jnp.float32),
            pltpu.VMEM((_SPW,), jnp.float32),
            pltpu.VMEM((_SPW * _D,), jnp.float32),
            pltpu.VMEM((_XCH, _D), jnp.float32),
        ],
    )
    return f(gl, wp, idx, x, rs)


def _epilogue_body(pool_ref, gsn_ref, mw_ref, mb_ref, o_ref):
    o_ref[...] = (
        jnp.dot(pool_ref[...], mw_ref[...], preferred_element_type=jnp.float32)
        + gsn_ref[...] * mb_ref[...]
    )


def _epilogue(pooled, gsn, msg_w, msg_b_row):
    rows = _NW * _SPW
    grid = (rows // _TB4,)
    return pl.pallas_call(
        _epilogue_body,
        grid=grid,
        in_specs=[
            pl.BlockSpec((_TB4, _D), lambda i: (i, 0)),
            pl.BlockSpec((_TB4, 1), lambda i: (i, 0)),
            pl.BlockSpec((_D, _D), lambda i: (0, 0)),
            pl.BlockSpec((1, _D), lambda i: (0, 0)),
        ],
        out_specs=pl.BlockSpec((_TB4, _D), lambda i: (i, 0)),
        out_shape=jax.ShapeDtypeStruct((rows, _D), jnp.float32),
    )(pooled, gsn, msg_w, msg_b_row)


def kernel(x, index, weights, gate_w, gate_b, msg_w, msg_b, pow_param):
    idx = index.astype(jnp.int32)
    gate_w_row = gate_w.reshape(1, _D)
    gl2, wp2 = _prologue(x, weights, gate_w_row, gate_b, pow_param)
    gl = gl2.reshape(_N)
    wp = wp2.reshape(_N)

    bounds = jnp.arange(_NW + 1, dtype=jnp.int32) * _SPW
    rs = jnp.searchsorted(idx, bounds, side="left").astype(jnp.int32)
    rs = jnp.pad(rs, (0, 48 - (_NW + 1)))

    pooled_flat, gsn = _sc_call(gl, wp, idx, x, rs)
    pooled = pooled_flat.reshape(_NW * _SPW, _D)
    out = _epilogue(pooled, gsn.reshape(-1, 1), msg_w, msg_b.reshape(1, _D))
    return out[:_S]


# speedup vs baseline: 4.6830x; 1.1914x over previous
import functools

import jax
import jax.numpy as jnp
from jax import lax
from jax.experimental import pallas as pl
from jax.experimental.pallas import tpu as pltpu
from jax.experimental.pallas import tpu_sc as plsc

_N = 320000
_D = 128
_S = 10000
_NW = 32
_SPW = 320
_CH = 256
_TB1 = 3200
_TB4 = 2048
_NEG = -3.0e38


def _prologue_body(x_ref, w_ref, gw_ref, gb_ref, p_ref, gl_ref, wp_ref):
    xv = x_ref[...]
    gw = gw_ref[...]
    gl_ref[...] = jnp.sum(xv * gw, axis=1, keepdims=True) + gb_ref[0]
    wp_ref[...] = jnp.power(w_ref[...], p_ref[0])


def _prologue(x, weights, gate_w_row, gate_b, pow_param):
    grid = (_N // _TB1,)
    return pl.pallas_call(
        _prologue_body,
        grid=grid,
        in_specs=[
            pl.BlockSpec((_TB1, _D), lambda i: (i, 0)),
            pl.BlockSpec((_TB1, 1), lambda i: (i, 0)),
            pl.BlockSpec((1, _D), lambda i: (0, 0)),
            pl.BlockSpec(memory_space=pltpu.SMEM),
            pl.BlockSpec(memory_space=pltpu.SMEM),
        ],
        out_specs=[
            pl.BlockSpec((_TB1, 1), lambda i: (i, 0)),
            pl.BlockSpec((_TB1, 1), lambda i: (i, 0)),
        ],
        out_shape=[
            jax.ShapeDtypeStruct((_N, 1), jnp.float32),
            jax.ShapeDtypeStruct((_N, 1), jnp.float32),
        ],
    )(x, weights, gate_w_row, gate_b, pow_param)


_LANE = 16


def _shift_up(v, k, fill):
    lanes = jax.lax.iota(jnp.int32, _LANE)
    src = jnp.maximum(lanes - k, 0)
    got = jnp.take_along_axis(v, src, axis=0)
    return jnp.where(lanes >= k, got, fill)


def _run_ends(li):
    lanes = jax.lax.iota(jnp.int32, _LANE)
    src = jnp.minimum(lanes + 1, _LANE - 1)
    nxt = jnp.take_along_axis(li, src, axis=0)
    return (nxt != li) | (lanes == _LANE - 1)


def _seg_scan_max(val, li):
    x = val
    for k in (1, 2, 4, 8):
        xs = _shift_up(x, k, _NEG)
        ls = _shift_up(li, k, -1)
        x = jnp.where(ls == li, jnp.maximum(x, xs), x)
    return x


def _seg_scan_sum(val, li):
    x = val
    for k in (1, 2, 4, 8):
        xs = _shift_up(x, k, 0.0)
        ls = _shift_up(li, k, -1)
        x = jnp.where(ls == li, x + xs, x)
    return x


def _sc_body(gl_hbm, wp_hbm, idx_hbm, x_hbm, rs_hbm,
             pooled_hbm, gsn_hbm,
             rsv, idxb, glb, wpb, livb, valb, mbuf, zbuf, gsnb, acc, xb, sem):
    w = lax.axis_index("s") * 2 + lax.axis_index("c")
    s0 = w * _SPW

    pltpu.sync_copy(rs_hbm, rsv)
    wv16 = jnp.full((_LANE,), w, jnp.int32)
    r0 = plsc.load_gather(rsv, [wv16])[0]
    r1 = plsc.load_gather(rsv, [wv16 + 1])[0]
    lo = r0 // _CH
    hi = jnp.maximum((r1 + _CH - 1) // _CH, lo)

    zero16 = jnp.zeros((_LANE,), jnp.float32)
    neg16 = jnp.full((_LANE,), _NEG, jnp.float32)

    def init_small(g, _):
        mbuf[pl.ds(g * 16, 16)] = neg16
        zbuf[pl.ds(g * 16, 16)] = zero16
        gsnb[pl.ds(g * 16, 16)] = zero16
        return 0
    lax.fori_loop(0, _SPW // 16, init_small, 0)

    def init_acc(g, _):
        acc[pl.ds(g * 16, 16)] = zero16
        return 0
    lax.fori_loop(0, _SPW * _D // 16, init_acc, 0)

    def fetch1(t):
        slot = lax.rem(t, 2)
        base = t * _CH
        pltpu.async_copy(gl_hbm.at[pl.ds(base, _CH)], glb.at[slot],
                         sem.at[slot])
        pltpu.async_copy(idx_hbm.at[pl.ds(base, _CH)], idxb.at[slot],
                         sem.at[slot])

    def wait1(t):
        slot = lax.rem(t, 2)
        base = t * _CH
        pltpu.make_async_copy(gl_hbm.at[pl.ds(base, _CH)], glb.at[slot],
                              sem.at[slot]).wait()
        pltpu.make_async_copy(idx_hbm.at[pl.ds(base, _CH)], idxb.at[slot],
                              sem.at[slot]).wait()

    @pl.when(lo < hi)
    def _():
        fetch1(lo)

    def chunk_max(t, _):
        slot = lax.rem(t, 2)
        wait1(t)

        @pl.when(t + 1 < hi)
        def _():
            fetch1(t + 1)

        def grp(g, _):
            iv = idxb[slot, pl.ds(g * 16, 16)]
            gv = glb[slot, pl.ds(g * 16, 16)]
            own = (iv >= s0) & (iv < s0 + _SPW)
            li = jnp.clip(iv - s0, 0, _SPW - 1)
            val = jnp.where(own, gv, _NEG)
            smax = _seg_scan_max(val, li)
            ends = _run_ends(li)
            cur = plsc.load_gather(mbuf, [li])
            plsc.store_scatter(mbuf, [li], jnp.maximum(cur, smax), mask=ends)
            return 0
        lax.fori_loop(0, _CH // 16, grp, 0)
        return 0
    lax.fori_loop(lo, hi, chunk_max, 0)

    def fetch2(t):
        slot = lax.rem(t, 2)
        base = t * _CH
        pltpu.async_copy(gl_hbm.at[pl.ds(base, _CH)], glb.at[slot],
                         sem.at[slot])
        pltpu.async_copy(idx_hbm.at[pl.ds(base, _CH)], idxb.at[slot],
                         sem.at[slot])
        pltpu.async_copy(wp_hbm.at[pl.ds(base, _CH)], wpb.at[slot],
                         sem.at[slot])
        pltpu.async_copy(x_hbm.at[pl.ds(base, _CH), :], xb.at[slot],
                         sem.at[slot])

    def wait2(t):
        slot = lax.rem(t, 2)
        base = t * _CH
        pltpu.make_async_copy(gl_hbm.at[pl.ds(base, _CH)], glb.at[slot],
                              sem.at[slot]).wait()
        pltpu.make_async_copy(idx_hbm.at[pl.ds(base, _CH)], idxb.at[slot],
                              sem.at[slot]).wait()
        pltpu.make_async_copy(wp_hbm.at[pl.ds(base, _CH)], wpb.at[slot],
                              sem.at[slot]).wait()
        pltpu.make_async_copy(x_hbm.at[pl.ds(base, _CH), :], xb.at[slot],
                              sem.at[slot]).wait()

    @pl.when(lo < hi)
    def _():
        fetch2(lo)

    def chunk_acc(t, _):
        slot = lax.rem(t, 2)
        wait2(t)

        @pl.when(t + 1 < hi)
        def _():
            fetch2(t + 1)

        def grp(g, _):
            iv = idxb[slot, pl.ds(g * 16, 16)]
            gv = glb[slot, pl.ds(g * 16, 16)]
            wvv = wpb[slot, pl.ds(g * 16, 16)]
            own = (iv >= s0) & (iv < s0 + _SPW)
            li = jnp.clip(iv - s0, 0, _SPW - 1)
            mg = plsc.load_gather(mbuf, [li])
            e = jnp.where(own, wvv * jnp.exp(gv - mg), 0.0)
            livb[pl.ds(g * 16, 16)] = li
            valb[pl.ds(g * 16, 16)] = e
            ssum = _seg_scan_sum(e, li)
            ends = _run_ends(li)
            cur = plsc.load_gather(zbuf, [li])
            plsc.store_scatter(zbuf, [li], cur + ssum, mask=ends)
            return 0
        lax.fori_loop(0, _CH // 16, grp, 0)

        def grpa(g, _):
            li16 = livb[pl.ds(g * 16, 16)]
            e16 = valb[pl.ds(g * 16, 16)]
            for lane in range(_LANE):
                li = li16[lane]
                e = e16[lane]
                off = li * _D
                row = g * 16 + lane
                for j in range(_D // 16):
                    xv = xb[slot, row, pl.ds(j * 16, 16)]
                    plsc.addupdate(acc.at[pl.ds(off + j * 16, 16)], xv * e)
            return 0
        lax.fori_loop(0, _CH // 16, grpa, 0)
        return 0
    lax.fori_loop(lo, hi, chunk_acc, 0)

    def fin(g, _):
        zv = zbuf[pl.ds(g * 16, 16)]
        inv = 1.0 / (zv + 1e-10)
        gsnb[pl.ds(g * 16, 16)] = zv * inv
        for lane in range(_LANE):
            ivl = inv[lane]
            off = (g * 16 + lane) * _D
            for j in range(_D // 16):
                sl = pl.ds(off + j * 16, 16)
                acc[sl] = acc[sl] * ivl
        return 0
    lax.fori_loop(0, _SPW // 16, fin, 0)

    pltpu.sync_copy(acc, pooled_hbm.at[pl.ds(w * _SPW * _D, _SPW * _D)])
    pltpu.sync_copy(gsnb, gsn_hbm.at[pl.ds(w * _SPW, _SPW)])


def _sc_call(gl, wp, idx, x, rs):
    mesh = plsc.VectorSubcoreMesh(core_axis_name="c", subcore_axis_name="s")
    f = pl.kernel(
        _sc_body,
        compiler_params=pltpu.CompilerParams(needs_layout_passes=False),
        out_type=[
            jax.ShapeDtypeStruct((_NW * _SPW * _D,), jnp.float32),
            jax.ShapeDtypeStruct((_NW * _SPW,), jnp.float32),
        ],
        mesh=mesh,
        scratch_types=[
            pltpu.VMEM((48,), jnp.int32),
            pltpu.VMEM((2, _CH), jnp.int32),
            pltpu.VMEM((2, _CH), jnp.float32),
            pltpu.VMEM((2, _CH), jnp.float32),
            pltpu.VMEM((_CH,), jnp.int32),
            pltpu.VMEM((_CH,), jnp.float32),
            pltpu.VMEM((_SPW,), jnp.float32),
            pltpu.VMEM((_SPW,), jnp.float32),
            pltpu.VMEM((_SPW,), jnp.float32),
            pltpu.VMEM((_SPW * _D,), jnp.float32),
            pltpu.VMEM((2, _CH, _D), jnp.float32),
            pltpu.SemaphoreType.DMA((2,)),
        ],
    )
    return f(gl, wp, idx, x, rs)


def _epilogue_body(pool_ref, gsn_ref, mw_ref, mb_ref, o_ref):
    o_ref[...] = (
        jnp.dot(pool_ref[...], mw_ref[...], preferred_element_type=jnp.float32)
        + gsn_ref[...] * mb_ref[...]
    )


def _epilogue(pooled, gsn, msg_w, msg_b_row):
    rows = _NW * _SPW
    grid = (rows // _TB4,)
    return pl.pallas_call(
        _epilogue_body,
        grid=grid,
        in_specs=[
            pl.BlockSpec((_TB4, _D), lambda i: (i, 0)),
            pl.BlockSpec((_TB4, 1), lambda i: (i, 0)),
            pl.BlockSpec((_D, _D), lambda i: (0, 0)),
            pl.BlockSpec((1, _D), lambda i: (0, 0)),
        ],
        out_specs=pl.BlockSpec((_TB4, _D), lambda i: (i, 0)),
        out_shape=jax.ShapeDtypeStruct((rows, _D), jnp.float32),
    )(pooled, gsn, msg_w, msg_b_row)


def kernel(x, index, weights, gate_w, gate_b, msg_w, msg_b, pow_param):
    idx = index.astype(jnp.int32)
    gate_w_row = gate_w.reshape(1, _D)
    gl2, wp2 = _prologue(x, weights, gate_w_row, gate_b, pow_param)
    gl = gl2.reshape(_N)
    wp = wp2.reshape(_N)

    bounds = jnp.arange(_NW + 1, dtype=jnp.int32) * _SPW
    rs = jnp.searchsorted(idx, bounds, side="left").astype(jnp.int32)
    rs = jnp.pad(rs, (0, 48 - (_NW + 1)))

    pooled_flat, gsn = _sc_call(gl, wp, idx, x, rs)
    pooled = pooled_flat.reshape(_NW * _SPW, _D)
    out = _epilogue(pooled, gsn.reshape(-1, 1), msg_w, msg_b.reshape(1, _D))
    return out[:_S]
